# baseline (speedup 1.0000x reference)
import functools

import jax
import jax.numpy as jnp
from jax import lax
from jax.experimental import pallas as pl
from jax.experimental.pallas import tpu as pltpu
from jax.experimental.pallas import tpu_sc as plsc

NUM_FACTORS = 16
NUM_ROWS = 1000000
BATCH = 16384
_ROWS_PER_BLOCK = 8
_NBLK = NUM_ROWS // _ROWS_PER_BLOCK

_NC, _NS = 2, 16
_NW = _NC * _NS
_BPW = BATCH // _NW
_CH = 128
_NCH = _BPW // _CH
_GROUP = 16


def _mf_body(user_hbm, item_hbm, uf_hbm, if_hbm, out_hbm,
             uidx_v, iidx_v, ublk_v, iblk_v, out_v,
             ub, vb, sems):
    wid = lax.axis_index("s") * _NC + lax.axis_index("c")
    base = wid * _BPW

    pltpu.sync_copy(user_hbm.at[pl.ds(base, _BPW)], uidx_v)
    pltpu.sync_copy(item_hbm.at[pl.ds(base, _BPW)], iidx_v)

    def blkstep(g, carry):
        o = g * _GROUP
        ublk_v[pl.ds(o, _GROUP)] = lax.shift_right_logical(
            uidx_v[pl.ds(o, _GROUP)], 3)
        iblk_v[pl.ds(o, _GROUP)] = lax.shift_right_logical(
            iidx_v[pl.ds(o, _GROUP)], 3)
        return carry

    lax.fori_loop(0, _BPW // _GROUP, blkstep, 0, unroll=False)

    def issue(c):
        buf = c % 2
        cu = pltpu.async_copy(
            uf_hbm.at[ublk_v.at[pl.ds(c * _CH, _CH)]], ub.at[buf], sems.at[2 * buf])
        cv = pltpu.async_copy(
            if_hbm.at[iblk_v.at[pl.ds(c * _CH, _CH)]], vb.at[buf], sems.at[2 * buf + 1])
        return cu, cv

    lane = lax.iota(jnp.int32, 16)
    diags = [(lane + k) & 15 for k in range(16)]

    def compute_chunk(c):
        buf = c % 2
        c0 = c * _CH

        def step(g, carry):
            o = g * _GROUP
            rows = lane + o
            usub = lax.shift_left(uidx_v[pl.ds(c0 + o, _GROUP)] & 7, 4)
            isub = lax.shift_left(iidx_v[pl.ds(c0 + o, _GROUP)] & 7, 4)
            acc = jnp.zeros((16,), jnp.float32)
            for k in range(16):
                du = plsc.load_gather(ub.at[buf], [rows, usub + diags[k]])
                dv = plsc.load_gather(vb.at[buf], [rows, isub + diags[k]])
                acc = acc + du * dv
            out_v[pl.ds(c0 + o, _GROUP)] = acc
            return carry

        lax.fori_loop(0, _CH // _GROUP, step, 0, unroll=False)

    cps = issue(0)
    for c in range(_NCH):
        cps[0].wait()
        cps[1].wait()
        if c + 1 < _NCH:
            cps = issue(c + 1)
        compute_chunk(c)

    pltpu.sync_copy(out_v, out_hbm.at[pl.ds(base, _BPW)])


@jax.jit
def _mf_call(user, item, uf_blocks, if_blocks):
    mesh = plsc.VectorSubcoreMesh(
        core_axis_name="c", subcore_axis_name="s",
        num_cores=_NC, num_subcores=_NS)
    return pl.kernel(
        _mf_body,
        out_type=jax.ShapeDtypeStruct((BATCH,), jnp.float32),
        mesh=mesh,
        compiler_params=pltpu.CompilerParams(
            needs_layout_passes=False, use_tc_tiling_on_sc=True),
        scratch_types=[
            pltpu.VMEM((_BPW,), jnp.int32),
            pltpu.VMEM((_BPW,), jnp.int32),
            pltpu.VMEM((_BPW,), jnp.int32),
            pltpu.VMEM((_BPW,), jnp.int32),
            pltpu.VMEM((_BPW,), jnp.float32),
            pltpu.VMEM((2, _CH, 128), jnp.float32),
            pltpu.VMEM((2, _CH, 128), jnp.float32),
            pltpu.SemaphoreType.DMA((4,)),
        ],
    )(user, item, uf_blocks, if_blocks)


def kernel(user, item, user_factors, item_factors):
    user = user.astype(jnp.int32)
    item = item.astype(jnp.int32)
    uf_blocks = user_factors.reshape(_NBLK, 128)
    if_blocks = item_factors.reshape(_NBLK, 128)
    return _mf_call(user, item, uf_blocks, if_blocks)
